# baseline (speedup 1.0000x reference)
import functools

import jax
import jax.numpy as jnp
import numpy as np
from jax import lax
from jax.experimental import pallas as pl
from jax.experimental.pallas import tpu as pltpu
from jax.experimental.pallas import tpu_sc as plsc

N = 10000
E = 160000
FEAT = 128
HID = 128
S = 4
NH = 8
L = 3
DH = HID // NH

_ROW_BLK = 1000


def _ln(t):
    m = t.mean(axis=-1, keepdims=True)
    v = t.var(axis=-1, keepdims=True)
    return (t - m) / jnp.sqrt(v + 1e-5)


_CHUNK = 64
_NCHUNK = E // _CHUNK
_NW = 32
_NPAD = 10240
_RPS = _NPAD // 16


def _edge_body(q_hbm, k_hbm, v_hbm, row_hbm, col_hbm, num_out, den_out,
               row_v, col_v, qr, kr, vr, exb, accn, accd, sem):
    cid = lax.axis_index("c")
    sid = lax.axis_index("s")
    wid = sid * 2 + cid

    zeros16 = jnp.zeros((16,), jnp.float32)

    def _zero_rows(i, _):
        for jj in range(HID // 16):
            qr[i, pl.ds(16 * jj, 16)] = zeros16
        exb[i, :] = zeros16
        return 0

    lax.fori_loop(0, _CHUNK, _zero_rows, 0)

    for t in range(_RPS // _CHUNK):
        pltpu.sync_copy(qr, accn.at[pl.ds(_RPS * sid + _CHUNK * t, _CHUNK), :])
        pltpu.sync_copy(exb, accd.at[pl.ds(_RPS * sid + _CHUNK * t, _CHUNK), :])
    plsc.subcore_barrier()

    rotidx = jnp.bitwise_and(lax.iota(jnp.int32, 16) + 8, 15)
    nloc = (_NCHUNK - wid + _NW - 1) // _NW

    def _chunk_body(i, carry):
        base = (wid + i * _NW) * _CHUNK
        pltpu.sync_copy(row_hbm.at[pl.ds(base, _CHUNK)], row_v)
        pltpu.sync_copy(col_hbm.at[pl.ds(base, _CHUNK)], col_v)
        cq = pltpu.async_copy(q_hbm.at[row_v], qr, sem)
        ck = pltpu.async_copy(k_hbm.at[col_v], kr, sem)
        cv = pltpu.async_copy(v_hbm.at[col_v], vr, sem)
        cq.wait()
        ck.wait()
        cv.wait()

        def _edge_loop(e, _):
            s = qr[e, pl.ds(0, 16)] * kr[e, pl.ds(0, 16)]
            for tt in range(1, 8):
                s = s + qr[e, pl.ds(16 * tt, 16)] * kr[e, pl.ds(16 * tt, 16)]
            exb[e, :] = s
            srot = plsc.load_gather(exb, [jnp.full((16,), e, jnp.int32), rotidx])
            ex16 = jnp.exp((s + srot) * 0.25)
            exb[e, :] = ex16
            for tt in range(8):
                vr[e, pl.ds(16 * tt, 16)] = vr[e, pl.ds(16 * tt, 16)] * ex16
            return 0

        lax.fori_loop(0, _CHUNK, _edge_loop, 0)
        pltpu.sync_copy(vr, accn.at[row_v], add=True)
        pltpu.sync_copy(exb, accd.at[row_v], add=True)
        return carry

    lax.fori_loop(0, nloc, _chunk_body, 0)
    plsc.subcore_barrier()
    pltpu.sync_copy(accn.at[pl.ds(_RPS * sid, _RPS), :],
                    num_out.at[cid, pl.ds(_RPS * sid, _RPS), :])
    pltpu.sync_copy(accd.at[pl.ds(_RPS * sid, _RPS), :],
                    den_out.at[cid, pl.ds(_RPS * sid, _RPS), :])


def _edge_pass(Q, Kn, Vn, row0, col0):
    mesh = plsc.VectorSubcoreMesh(core_axis_name="c", subcore_axis_name="s")
    f = functools.partial(
        pl.kernel,
        out_type=[
            jax.ShapeDtypeStruct((2, _NPAD, HID), jnp.float32),
            jax.ShapeDtypeStruct((2, _NPAD, 16), jnp.float32),
        ],
        mesh=mesh,
        compiler_params=pltpu.CompilerParams(
            needs_layout_passes=False, use_tc_tiling_on_sc=False),
        scratch_types=[
            pltpu.VMEM((_CHUNK,), jnp.int32),
            pltpu.VMEM((_CHUNK,), jnp.int32),
            pltpu.VMEM((_CHUNK, HID), jnp.float32),
            pltpu.VMEM((_CHUNK, HID), jnp.float32),
            pltpu.VMEM((_CHUNK, HID), jnp.float32),
            pltpu.VMEM((_CHUNK, 16), jnp.float32),
            pltpu.VMEM_SHARED((_NPAD, HID), jnp.float32),
            pltpu.VMEM_SHARED((_NPAD, 16), jnp.float32),
            pltpu.SemaphoreType.DMA,
        ],
    )(_edge_body)
    return f(Q, Kn, Vn, row0, col0)


def _agg_ln_body(num_ref, den_ref, h_ref, wo_ref, out_ref):
    agg = num_ref[...] / (den_ref[...] + 1e-16)
    y = jnp.dot(agg, wo_ref[...], preferred_element_type=jnp.float32) + h_ref[...]
    m = y.mean(axis=-1, keepdims=True)
    v = ((y - m) ** 2).mean(axis=-1, keepdims=True)
    out_ref[...] = (y - m) / jnp.sqrt(v + 1e-5)


def _agg_ln(num, den128, h, Wo):
    grid = (N // _ROW_BLK,)
    return pl.pallas_call(
        _agg_ln_body,
        grid=grid,
        in_specs=[
            pl.BlockSpec((_ROW_BLK, HID), lambda i: (i, 0)),
            pl.BlockSpec((_ROW_BLK, HID), lambda i: (i, 0)),
            pl.BlockSpec((_ROW_BLK, HID), lambda i: (i, 0)),
            pl.BlockSpec((HID, HID), lambda i: (0, 0)),
        ],
        out_specs=pl.BlockSpec((_ROW_BLK, HID), lambda i: (i, 0)),
        out_shape=jax.ShapeDtypeStruct((N, HID), jnp.float32),
    )(num, den128, h, Wo)


def kernel(x, edge_index, batch, fl_W, fl_b, si_Wq, si_Wk, si_Wv, conv_Wq, conv_Wk, conv_Wv, conv_Wo, conv_relk, conv_relv, sa_Wq, sa_Wk, sa_Wv):
    h = jax.nn.relu(x @ fl_W + fl_b)
    row0, col0 = edge_index[0], edge_index[1]
    seed = h.mean(axis=0, keepdims=True)
    q0 = (seed @ si_Wq).reshape(S, HID)
    k0 = (h @ si_Wk).reshape(N, S, HID)
    v0 = (h @ si_Wv).reshape(N, S, HID)
    lg0 = jnp.einsum('nsd,sd->ns', k0, q0) / np.sqrt(HID)
    e0 = jnp.exp(lg0)
    a0 = e0 / e0.sum(axis=0, keepdims=True)
    stars = _ln(jnp.einsum('ns,nsd->sd', a0, v0).reshape(-1)).reshape(S, HID)
    perm = np.array([j * DH + t for t in range(DH) for j in range(NH)])
    for l in range(L):
        Qp = h @ conv_Wq[l][:, perm]
        Knp = h @ conv_Wk[l][:, perm] + conv_relk[l, 0][perm]
        Vnp = h @ conv_Wv[l][:, perm] + conv_relv[l, 0][perm]
        Ksp = stars @ conv_Wk[l][:, perm] + conv_relk[l, 0][perm]
        Vsp = stars @ conv_Wv[l][:, perm] + conv_relv[l, 0][perm]
        ex_self = jnp.exp((Qp * Knp).reshape(N, DH, NH).sum(axis=1) / np.sqrt(DH))
        lg_star = jnp.einsum('ntj,stj->nsj', Qp.reshape(N, DH, NH),
                             Ksp.reshape(S, DH, NH)) / np.sqrt(DH)
        ex_star = jnp.exp(lg_star)
        den = ex_self + ex_star.sum(axis=1)
        num = (jnp.tile(ex_self, (1, DH)) * Vnp
               + jnp.einsum('nsj,stj->ntj', ex_star,
                            Vsp.reshape(S, DH, NH)).reshape(N, HID))
        num_sc, den_sc = _edge_pass(Qp, Knp, Vnp, row0, col0)
        den = den + den_sc[0, :N, :NH] + den_sc[1, :N, :NH]
        num = num + num_sc[0, :N] + num_sc[1, :N]
        den128 = jnp.tile(den, (1, DH))
        h = _agg_ln(num, den128, h, conv_Wo[l][perm, :])
        src = jnp.concatenate([h, stars], axis=0)
        sq = (stars @ sa_Wq[l]).reshape(S, NH, DH)
        sk = (src @ sa_Wk[l]).reshape(N + S, NH, DH)
        sv = (src @ sa_Wv[l]).reshape(N + S, NH, DH)
        lg2 = jnp.einsum('shd,mhd->shm', sq, sk) / np.sqrt(DH)
        e2 = jnp.exp(lg2)
        so = (jnp.einsum('shm,mhd->shd', e2, sv) / e2.sum(-1, keepdims=True)).reshape(S, HID)
        stars = _ln(so + stars)
    return (h, stars.reshape(1, S, HID), h)


# speedup vs baseline: 1.2365x; 1.2365x over previous
import functools

import jax
import jax.numpy as jnp
import numpy as np
from jax import lax
from jax.experimental import pallas as pl
from jax.experimental.pallas import tpu as pltpu
from jax.experimental.pallas import tpu_sc as plsc

N = 10000
E = 160000
FEAT = 128
HID = 128
S = 4
NH = 8
L = 3
DH = HID // NH

_ROW_BLK = 1000


def _ln(t):
    m = t.mean(axis=-1, keepdims=True)
    v = t.var(axis=-1, keepdims=True)
    return (t - m) / jnp.sqrt(v + 1e-5)


_CHUNK = 64
_NCHUNK = E // _CHUNK
_NW = 32
_NPAD = 10240
_RPS = _NPAD // 16


def _edge_body(q_hbm, k_hbm, v_hbm, row_hbm, col_hbm, num_out, den_out,
               row_v2, col_v2, qr, kr, vr2, exb2, accn, accd, semi, semg, sems):
    cid = lax.axis_index("c")
    sid = lax.axis_index("s")
    wid = sid * 2 + cid

    zeros16 = jnp.zeros((16,), jnp.float32)

    def _zero_rows(i, _):
        for jj in range(HID // 16):
            qr[i, pl.ds(16 * jj, 16)] = zeros16
        exb2[0, i, :] = zeros16
        return 0

    lax.fori_loop(0, _CHUNK, _zero_rows, 0)

    for t in range(_RPS // _CHUNK):
        pltpu.sync_copy(qr, accn.at[pl.ds(_RPS * sid + _CHUNK * t, _CHUNK), :])
        pltpu.sync_copy(exb2.at[0], accd.at[pl.ds(_RPS * sid + _CHUNK * t, _CHUNK), :])
    plsc.subcore_barrier()

    rotidx = jnp.bitwise_and(lax.iota(jnp.int32, 16) + 8, 15)
    nloc = (_NCHUNK - wid + _NW - 1) // _NW

    pltpu.async_copy(row_hbm.at[pl.ds(wid * _CHUNK, _CHUNK)], row_v2.at[0], semi)
    pltpu.async_copy(col_hbm.at[pl.ds(wid * _CHUNK, _CHUNK)], col_v2.at[0], semi)

    def _chunk_body(i, carry):
        slot = jnp.bitwise_and(i, 1)
        pslot = 1 - slot
        row_v = row_v2.at[slot]
        col_v = col_v2.at[slot]
        vr = vr2.at[slot]
        exb = exb2.at[slot]

        @pl.when(i > 0)
        def _():
            pltpu.async_copy(vr2.at[pslot], accn.at[row_v2.at[pslot]], sems,
                             add=True)
            pltpu.async_copy(exb2.at[pslot], accd.at[row_v2.at[pslot]], sems,
                             add=True)

        pltpu.make_async_copy(row_hbm.at[pl.ds(0, _CHUNK)], row_v, semi).wait()
        pltpu.make_async_copy(col_hbm.at[pl.ds(0, _CHUNK)], col_v, semi).wait()
        cq = pltpu.async_copy(q_hbm.at[row_v], qr, semg)
        ck = pltpu.async_copy(k_hbm.at[col_v], kr, semg)
        cvv = pltpu.async_copy(v_hbm.at[col_v], vr, semg)

        @pl.when(i + 1 < nloc)
        def _():
            nbase = (wid + (i + 1) * _NW) * _CHUNK
            pltpu.async_copy(row_hbm.at[pl.ds(nbase, _CHUNK)],
                             row_v2.at[pslot], semi)
            pltpu.async_copy(col_hbm.at[pl.ds(nbase, _CHUNK)],
                             col_v2.at[pslot], semi)

        cq.wait()
        ck.wait()
        cvv.wait()

        def _edge_loop(e, _):
            s = qr[e, pl.ds(0, 16)] * kr[e, pl.ds(0, 16)]
            for tt in range(1, 8):
                s = s + qr[e, pl.ds(16 * tt, 16)] * kr[e, pl.ds(16 * tt, 16)]
            exb[e, :] = s
            srot = plsc.load_gather(exb, [jnp.full((16,), e, jnp.int32), rotidx])
            ex16 = jnp.exp((s + srot) * 0.25)
            exb[e, :] = ex16
            for tt in range(8):
                vr[e, pl.ds(16 * tt, 16)] = vr[e, pl.ds(16 * tt, 16)] * ex16
            return 0

        lax.fori_loop(0, _CHUNK, _edge_loop, 0)

        @pl.when(i > 0)
        def _():
            pltpu.make_async_copy(vr2.at[pslot], accn.at[pl.ds(0, _CHUNK)],
                                  sems).wait()
            pltpu.make_async_copy(exb2.at[pslot], accd.at[pl.ds(0, _CHUNK)],
                                  sems).wait()
        return carry

    lax.fori_loop(0, nloc, _chunk_body, 0)

    @pl.when(nloc > 0)
    def _():
        lslot = jnp.bitwise_and(nloc - 1, 1)
        pltpu.async_copy(vr2.at[lslot], accn.at[row_v2.at[lslot]], sems,
                         add=True)
        pltpu.async_copy(exb2.at[lslot], accd.at[row_v2.at[lslot]], sems,
                         add=True)
        pltpu.make_async_copy(vr2.at[lslot], accn.at[pl.ds(0, _CHUNK)],
                              sems).wait()
        pltpu.make_async_copy(exb2.at[lslot], accd.at[pl.ds(0, _CHUNK)],
                              sems).wait()

    plsc.subcore_barrier()
    pltpu.sync_copy(accn.at[pl.ds(_RPS * sid, _RPS), :],
                    num_out.at[cid, pl.ds(_RPS * sid, _RPS), :])
    pltpu.sync_copy(accd.at[pl.ds(_RPS * sid, _RPS), :],
                    den_out.at[cid, pl.ds(_RPS * sid, _RPS), :])


def _edge_pass(Q, Kn, Vn, row0, col0):
    mesh = plsc.VectorSubcoreMesh(core_axis_name="c", subcore_axis_name="s")
    f = functools.partial(
        pl.kernel,
        out_type=[
            jax.ShapeDtypeStruct((2, _NPAD, HID), jnp.float32),
            jax.ShapeDtypeStruct((2, _NPAD, 16), jnp.float32),
        ],
        mesh=mesh,
        compiler_params=pltpu.CompilerParams(
            needs_layout_passes=False, use_tc_tiling_on_sc=False),
        scratch_types=[
            pltpu.VMEM((2, _CHUNK), jnp.int32),
            pltpu.VMEM((2, _CHUNK), jnp.int32),
            pltpu.VMEM((_CHUNK, HID), jnp.float32),
            pltpu.VMEM((_CHUNK, HID), jnp.float32),
            pltpu.VMEM((2, _CHUNK, HID), jnp.float32),
            pltpu.VMEM((2, _CHUNK, 16), jnp.float32),
            pltpu.VMEM_SHARED((_NPAD, HID), jnp.float32),
            pltpu.VMEM_SHARED((_NPAD, 16), jnp.float32),
            pltpu.SemaphoreType.DMA,
            pltpu.SemaphoreType.DMA,
            pltpu.SemaphoreType.DMA,
        ],
    )(_edge_body)
    return f(Q, Kn, Vn, row0, col0)


def _agg_ln_body(num_ref, den_ref, h_ref, wo_ref, out_ref):
    agg = num_ref[...] / (den_ref[...] + 1e-16)
    y = jnp.dot(agg, wo_ref[...], preferred_element_type=jnp.float32) + h_ref[...]
    m = y.mean(axis=-1, keepdims=True)
    v = ((y - m) ** 2).mean(axis=-1, keepdims=True)
    out_ref[...] = (y - m) / jnp.sqrt(v + 1e-5)


def _agg_ln(num, den128, h, Wo):
    grid = (N // _ROW_BLK,)
    return pl.pallas_call(
        _agg_ln_body,
        grid=grid,
        in_specs=[
            pl.BlockSpec((_ROW_BLK, HID), lambda i: (i, 0)),
            pl.BlockSpec((_ROW_BLK, HID), lambda i: (i, 0)),
            pl.BlockSpec((_ROW_BLK, HID), lambda i: (i, 0)),
            pl.BlockSpec((HID, HID), lambda i: (0, 0)),
        ],
        out_specs=pl.BlockSpec((_ROW_BLK, HID), lambda i: (i, 0)),
        out_shape=jax.ShapeDtypeStruct((N, HID), jnp.float32),
    )(num, den128, h, Wo)


def kernel(x, edge_index, batch, fl_W, fl_b, si_Wq, si_Wk, si_Wv, conv_Wq, conv_Wk, conv_Wv, conv_Wo, conv_relk, conv_relv, sa_Wq, sa_Wk, sa_Wv):
    h = jax.nn.relu(x @ fl_W + fl_b)
    row0, col0 = edge_index[0], edge_index[1]
    seed = h.mean(axis=0, keepdims=True)
    q0 = (seed @ si_Wq).reshape(S, HID)
    k0 = (h @ si_Wk).reshape(N, S, HID)
    v0 = (h @ si_Wv).reshape(N, S, HID)
    lg0 = jnp.einsum('nsd,sd->ns', k0, q0) / np.sqrt(HID)
    e0 = jnp.exp(lg0)
    a0 = e0 / e0.sum(axis=0, keepdims=True)
    stars = _ln(jnp.einsum('ns,nsd->sd', a0, v0).reshape(-1)).reshape(S, HID)
    perm = np.array([j * DH + t for t in range(DH) for j in range(NH)])
    for l in range(L):
        Qp = h @ conv_Wq[l][:, perm]
        Knp = h @ conv_Wk[l][:, perm] + conv_relk[l, 0][perm]
        Vnp = h @ conv_Wv[l][:, perm] + conv_relv[l, 0][perm]
        Ksp = stars @ conv_Wk[l][:, perm] + conv_relk[l, 0][perm]
        Vsp = stars @ conv_Wv[l][:, perm] + conv_relv[l, 0][perm]
        ex_self = jnp.exp((Qp * Knp).reshape(N, DH, NH).sum(axis=1) / np.sqrt(DH))
        lg_star = jnp.einsum('ntj,stj->nsj', Qp.reshape(N, DH, NH),
                             Ksp.reshape(S, DH, NH)) / np.sqrt(DH)
        ex_star = jnp.exp(lg_star)
        den = ex_self + ex_star.sum(axis=1)
        num = (jnp.tile(ex_self, (1, DH)) * Vnp
               + jnp.einsum('nsj,stj->ntj', ex_star,
                            Vsp.reshape(S, DH, NH)).reshape(N, HID))
        num_sc, den_sc = _edge_pass(Qp, Knp, Vnp, row0, col0)
        den = den + den_sc[0, :N, :NH] + den_sc[1, :N, :NH]
        num = num + num_sc[0, :N] + num_sc[1, :N]
        den128 = jnp.tile(den, (1, DH))
        h = _agg_ln(num, den128, h, conv_Wo[l][perm, :])
        src = jnp.concatenate([h, stars], axis=0)
        sq = (stars @ sa_Wq[l]).reshape(S, NH, DH)
        sk = (src @ sa_Wk[l]).reshape(N + S, NH, DH)
        sv = (src @ sa_Wv[l]).reshape(N + S, NH, DH)
        lg2 = jnp.einsum('shd,mhd->shm', sq, sk) / np.sqrt(DH)
        e2 = jnp.exp(lg2)
        so = (jnp.einsum('shm,mhd->shd', e2, sv) / e2.sum(-1, keepdims=True)).reshape(S, HID)
        stars = _ln(so + stars)
    return (h, stars.reshape(1, S, HID), h)
